# baseline (speedup 1.0000x reference)
import functools

import jax
import jax.numpy as jnp
from jax import lax
from jax.experimental import pallas as pl
from jax.experimental.pallas import tpu as pltpu
from jax.experimental.pallas import tpu_sc as plsc

_TOPK = (1, 5)
_THR = 0.0

_R_SC = 512
_SC_CHUNK = 20000
_SC_UNROLL = 10


def _sc_gather_slivers(pred128, t32, num_rows, num_cols):
    info = plsc.get_sparse_core_info()
    nw = info.num_cores * info.num_subcores
    b_per_w = num_rows // nw
    assert num_rows % nw == 0 and b_per_w % 16 == 0

    mesh = plsc.VectorSubcoreMesh(core_axis_name="c", subcore_axis_name="s")

    @functools.partial(
        pl.kernel,
        mesh=mesh,
        out_type=jax.ShapeDtypeStruct((num_rows, 128), jnp.float32),
        scratch_types=[
            pltpu.VMEM((b_per_w,), jnp.int32),
            pltpu.VMEM((b_per_w,), jnp.int32),
            pltpu.VMEM((b_per_w, 128), jnp.float32),
            pltpu.SemaphoreType.DMA,
        ],
    )
    def gather_kernel(pred_hbm, t_hbm, out_hbm, t_v, idx_v, rows_v, sem):
        wid = lax.axis_index("s") * info.num_cores + lax.axis_index("c")
        base = wid * b_per_w
        pltpu.sync_copy(t_hbm.at[pl.ds(base, b_per_w)], t_v)
        for k in range(b_per_w // 16):
            t16 = t_v[pl.ds(k * 16, 16)]
            rows16 = base + k * 16 + lax.iota(jnp.int32, 16)
            flat16 = rows16 * num_cols + t16
            idx_v[pl.ds(k * 16, 16)] = flat16 >> 7
        pltpu.async_copy(pred_hbm.at[idx_v], rows_v, sem).wait()
        pltpu.sync_copy(rows_v, out_hbm.at[pl.ds(base, b_per_w)])

    return gather_kernel(pred128, t32)


def _tc_extract_s(slivers, t2d, num_rows, num_cols):
    def body(sliv_ref, t_ref, out_ref):
        sliv = sliv_ref[...]
        t = t_ref[...]
        row = lax.broadcasted_iota(jnp.int32, (num_rows, 1), 0)
        off = (row * num_cols + t) & 127
        lane = lax.broadcasted_iota(jnp.int32, (num_rows, 128), 1)
        picked = jnp.where(lane == off, sliv, 0.0)
        out_ref[...] = jnp.sum(picked, axis=1, keepdims=True)

    return pl.pallas_call(
        body,
        out_shape=jax.ShapeDtypeStruct((num_rows, 1), jnp.float32),
    )(slivers, t2d)


def _sc_count_ranks(pred_flat, s1d, t32, row0, r_sc, num_cols):
    info = plsc.get_sparse_core_info()
    nw = info.num_cores * info.num_subcores
    rpw = r_sc // nw
    assert r_sc % nw == 0 and rpw % 8 == 0 and row0 % 8 == 0
    rpad = ((rpw + 15) // 16) * 16
    w = _SC_CHUNK
    nch = num_cols // w
    assert num_cols % w == 0 and w % (16 * _SC_UNROLL) == 0
    steps = w // (16 * _SC_UNROLL)

    mesh = plsc.VectorSubcoreMesh(core_axis_name="c", subcore_axis_name="s")

    @functools.partial(
        pl.kernel,
        mesh=mesh,
        out_type=jax.ShapeDtypeStruct((r_sc, 16), jnp.int32),
        scratch_types=[
            pltpu.VMEM((rpad,), jnp.float32),
            pltpu.VMEM((rpad,), jnp.int32),
            pltpu.VMEM((rpad, 16), jnp.int32),
            pltpu.VMEM((w,), jnp.float32),
            pltpu.VMEM((w,), jnp.float32),
            pltpu.SemaphoreType.DMA,
            pltpu.SemaphoreType.DMA,
        ],
    )
    def count_kernel(pred_hbm, s_hbm, t_hbm, out_hbm,
                     s_v, t_v, rank_v, buf0, buf1, sem0, sem1):
        wid = lax.axis_index("s") * info.num_cores + lax.axis_index("c")
        base = row0 + wid * rpw
        pltpu.sync_copy(s_hbm.at[pl.ds(base, rpw)], s_v.at[pl.ds(0, rpw)])
        pltpu.sync_copy(t_hbm.at[pl.ds(base, rpw)], t_v.at[pl.ds(0, rpw)])
        bufs = (buf0, buf1)
        sems = (sem0, sem1)
        lane_iota = lax.iota(jnp.int32, 16)

        for g in range((rpw + 15) // 16):
            s16 = s_v[pl.ds(g * 16, 16)]
            t16 = t_v[pl.ds(g * 16, 16)]
            for lr in range(min(16, rpw - g * 16)):
                r = g * 16 + lr
                s_spl = lax.broadcast(s16[lr], (16,))
                t_spl = lax.broadcast(t16[lr], (16,))
                rowbase = (base + r) * num_cols

                def chunk_start(ch, slot):
                    return pltpu.async_copy(
                        pred_hbm.at[pl.ds(rowbase + ch * w, w)],
                        bufs[slot], sems[slot])

                acc = jnp.zeros((16,), jnp.int32)
                col = lane_iota
                h = chunk_start(0, 0)
                for ch in range(nch):
                    h_next = (chunk_start(ch + 1, (ch + 1) % 2)
                              if ch + 1 < nch else None)
                    h.wait()
                    buf = bufs[ch % 2]

                    def step(j, carry):
                        a, cc = carry
                        off = j * (16 * _SC_UNROLL)
                        for u in range(_SC_UNROLL):
                            v = buf[pl.ds(off + u * 16, 16)]
                            gt = v > s_spl
                            tie = (v == s_spl) & (cc < t_spl)
                            a = a + jnp.where(gt, 1, 0) + jnp.where(tie, 1, 0)
                            cc = cc + 16
                        return a, cc

                    acc, col = lax.fori_loop(0, steps, step, (acc, col))
                    h = h_next
                rank_v[r, :] = acc
        pltpu.sync_copy(rank_v.at[pl.ds(0, rpw)],
                        out_hbm.at[pl.ds(wid * rpw, rpw)])

    return count_kernel(pred_flat, s1d, t32)


def _tc_rank_scan(pred, t2d, s2d, r_tc, num_cols, cb):
    nb = (num_cols + cb - 1) // cb

    def body(pred_ref, t_ref, s_ref, out_ref, acc_ref):
        c = pl.program_id(0)

        @pl.when(c == 0)
        def _init():
            acc_ref[...] = jnp.zeros_like(acc_ref)

        v = pred_ref[...]
        s = s_ref[...]
        t = t_ref[...]
        col0 = c * cb
        rel = lax.broadcasted_iota(jnp.int32, (r_tc, cb), 1)
        gt = (v > s) & (rel < (num_cols - col0))
        eqb = (v == s) & (rel < (t - col0))
        cnt = (gt | eqb).astype(jnp.int32)
        part = cnt[:, 0:128]
        for k in range(1, cb // 128):
            part = part + cnt[:, k * 128:(k + 1) * 128]
        acc_ref[...] += part

        @pl.when(c == nb - 1)
        def _fin():
            out_ref[...] = jnp.sum(acc_ref[...], axis=1, keepdims=True)

    return pl.pallas_call(
        body,
        grid=(nb,),
        in_specs=[
            pl.BlockSpec((r_tc, cb), lambda c: (0, c)),
            pl.BlockSpec((r_tc, 1), lambda c: (0, 0)),
            pl.BlockSpec((r_tc, 1), lambda c: (0, 0)),
        ],
        out_specs=pl.BlockSpec((r_tc, 1), lambda c: (0, 0)),
        out_shape=jax.ShapeDtypeStruct((r_tc, 1), jnp.int32),
        scratch_shapes=[pltpu.VMEM((r_tc, 128), jnp.int32)],
        compiler_params=pltpu.CompilerParams(
            dimension_semantics=("arbitrary",)),
    )(pred, t2d, s2d)


def _tc_finalize(rank_tc, rank_sc, s2d, r_tc, num_rows):
    def body(rtc_ref, rsc_ref, s_ref, out_ref):
        rtc = rtc_ref[...]
        rsc = jnp.sum(rsc_ref[...], axis=1, keepdims=True)
        s = s_ref[...]
        ok_tc = s[0:r_tc, :] > _THR
        ok_sc = s[r_tc:, :] > _THR
        t1 = (jnp.sum(((rtc < _TOPK[0]) & ok_tc).astype(jnp.float32))
              + jnp.sum(((rsc < _TOPK[0]) & ok_sc).astype(jnp.float32)))
        t5 = (jnp.sum(((rtc < _TOPK[1]) & ok_tc).astype(jnp.float32))
              + jnp.sum(((rsc < _TOPK[1]) & ok_sc).astype(jnp.float32)))
        lanes = lax.broadcasted_iota(jnp.int32, (1, 2), 1)
        out_ref[...] = jnp.where(lanes == 0, t1, t5) * (100.0 / num_rows)

    return pl.pallas_call(
        body,
        out_shape=jax.ShapeDtypeStruct((1, 2), jnp.float32),
    )(rank_tc, rank_sc, s2d)


def kernel(pred, target):
    num_rows, num_cols = pred.shape
    r_sc = _R_SC
    r_tc = num_rows - r_sc
    t32 = target.astype(jnp.int32)
    pred128 = pred.reshape(num_rows * num_cols // 128, 128)
    slivers = _sc_gather_slivers(pred128, t32, num_rows, num_cols)
    s2d = _tc_extract_s(slivers, t32.reshape(num_rows, 1),
                        num_rows, num_cols)
    rank_sc = _sc_count_ranks(pred.reshape(-1), s2d.reshape(-1), t32,
                              r_tc, r_sc, num_cols)
    rank_tc = _tc_rank_scan(pred, t32[:r_tc].reshape(r_tc, 1),
                            s2d[:r_tc], r_tc, num_cols, cb=2048)
    return _tc_finalize(rank_tc, rank_sc, s2d, r_tc, num_rows).reshape(2)


# speedup vs baseline: 3.2581x; 3.2581x over previous
import jax
import jax.numpy as jnp
from jax import lax
from jax.experimental import pallas as pl
from jax.experimental.pallas import tpu as pltpu

_TOPK = (1, 5)
_THR = 0.0
_G = 16


def _tc_gather_scores(pred, t32, num_rows, num_cols):
    assert num_rows % _G == 0

    def body(t_ref, *refs):
        i = pl.program_id(0)
        out_ref = refs[_G]
        rowg = lax.broadcasted_iota(jnp.int32, (_G, 1), 0)
        row8 = lax.broadcasted_iota(jnp.int32, (8, 128), 0)
        lane = lax.broadcasted_iota(jnp.int32, (8, 128), 1)
        acc = jnp.zeros((_G, 1), jnp.float32)
        for k in range(_G):
            r = i * _G + k
            off = t_ref[r] & 127
            v = refs[k][...]
            picked = jnp.where((row8 == (r & 7)) & (lane == off), v, 0.0)
            acc = acc + jnp.where(rowg == k, jnp.sum(picked), 0.0)
        out_ref[...] = acc

    def mk_spec(k):
        return pl.BlockSpec(
            (8, 128),
            lambda i, tref, k=k: ((i * _G + k) // 8, tref[i * _G + k] // 128))

    return pl.pallas_call(
        body,
        grid_spec=pltpu.PrefetchScalarGridSpec(
            num_scalar_prefetch=1,
            grid=(num_rows // _G,),
            in_specs=[mk_spec(k) for k in range(_G)],
            out_specs=pl.BlockSpec((_G, 1), lambda i, tref: (i, 0)),
        ),
        out_shape=jax.ShapeDtypeStruct((num_rows, 1), jnp.float32),
        compiler_params=pltpu.CompilerParams(
            dimension_semantics=("arbitrary",)),
    )(t32, *([pred] * _G))


def _tc_rank_scan(pred, t2d, s2d, num_rows, num_cols, cb):
    nb = (num_cols + cb - 1) // cb

    def body(pred_ref, t_ref, s_ref, out_ref, acc_ref):
        c = pl.program_id(0)

        @pl.when(c == 0)
        def _init():
            acc_ref[...] = jnp.zeros_like(acc_ref)

        v = pred_ref[...]
        s = s_ref[...]
        t = t_ref[...]
        col0 = c * cb
        rel = lax.broadcasted_iota(jnp.int32, (num_rows, cb), 1)
        eqb = (v == s) & (rel < (t - col0))

        @pl.when(c < nb - 1)
        def _mid():
            cnt = ((v > s) | eqb).astype(jnp.int32)
            part = cnt[:, 0:128]
            for k in range(1, cb // 128):
                part = part + cnt[:, k * 128:(k + 1) * 128]
            acc_ref[...] += part

        @pl.when(c == nb - 1)
        def _last():
            gt = (v > s) & (rel < (num_cols - col0))
            cnt = (gt | eqb).astype(jnp.int32)
            part = cnt[:, 0:128]
            for k in range(1, cb // 128):
                part = part + cnt[:, k * 128:(k + 1) * 128]
            acc_ref[...] += part

            rank = jnp.sum(acc_ref[...], axis=1, keepdims=True)
            ok = s > _THR
            t1 = jnp.sum(((rank < _TOPK[0]) & ok).astype(jnp.float32))
            t5 = jnp.sum(((rank < _TOPK[1]) & ok).astype(jnp.float32))
            lanes = lax.broadcasted_iota(jnp.int32, (1, 2), 1)
            out_ref[...] = jnp.where(lanes == 0, t1, t5) * (100.0 / num_rows)

    return pl.pallas_call(
        body,
        grid=(nb,),
        in_specs=[
            pl.BlockSpec((num_rows, cb), lambda c: (0, c)),
            pl.BlockSpec((num_rows, 1), lambda c: (0, 0)),
            pl.BlockSpec((num_rows, 1), lambda c: (0, 0)),
        ],
        out_specs=pl.BlockSpec((1, 2), lambda c: (0, 0)),
        out_shape=jax.ShapeDtypeStruct((1, 2), jnp.float32),
        scratch_shapes=[pltpu.VMEM((num_rows, 128), jnp.int32)],
        compiler_params=pltpu.CompilerParams(
            dimension_semantics=("arbitrary",)),
    )(pred, t2d, s2d)


def kernel(pred, target):
    num_rows, num_cols = pred.shape
    t32 = target.astype(jnp.int32)
    s2d = _tc_gather_scores(pred, t32, num_rows, num_cols)
    out = _tc_rank_scan(pred, t32.reshape(num_rows, 1), s2d,
                        num_rows, num_cols, cb=2048)
    return out.reshape(2)
